# Initial kernel scaffold; baseline (speedup 1.0000x reference)
import functools

import jax
import jax.numpy as jnp
from jax import lax
from jax.experimental import pallas as pl
from jax.experimental.pallas import tpu as pltpu
from jax.experimental.pallas import tpu_sc as plsc

N = 10000
E = 320000
D = 128
G = 64
NC = 2
NS = 16
NW = NC * NS
CHUNK = 128
EP = 331776
NCHUNK = EP // (NW * CHUNK)
PER_W = EP // NW

_MESH = plsc.VectorSubcoreMesh(core_axis_name="c", subcore_axis_name="s")


@functools.partial(
    pl.kernel,
    out_type=jax.ShapeDtypeStruct((NC, N), jnp.float32),
    mesh=_MESH,
    scratch_types=[
        pltpu.VMEM((CHUNK,), jnp.int32),
        pltpu.VMEM((CHUNK,), jnp.float32),
        pltpu.VMEM_SHARED((N,), jnp.float32),
    ],
)
def _deg_kernel(dst_hbm, ew_hbm, zero_hbm, out_hbm, idx_v, val_v, acc):
    c = lax.axis_index("c")
    s = lax.axis_index("s")
    wid = s * NC + c

    @pl.when(s == 0)
    def _():
        pltpu.sync_copy(zero_hbm, acc)

    plsc.subcore_barrier()

    def body(i, carry):
        base = wid * PER_W + i * CHUNK
        pltpu.sync_copy(dst_hbm.at[pl.ds(base, CHUNK)], idx_v)
        pltpu.sync_copy(ew_hbm.at[pl.ds(base, CHUNK)], val_v)
        pltpu.sync_copy(val_v, acc.at[idx_v], add=True)
        return carry

    lax.fori_loop(0, NCHUNK, body, 0)
    plsc.subcore_barrier()

    @pl.when(s == 0)
    def _():
        pltpu.sync_copy(acc, out_hbm.at[c])


@functools.partial(
    pl.kernel,
    out_type=jax.ShapeDtypeStruct((NC, N, D), jnp.float32),
    mesh=_MESH,
    scratch_types=[
        pltpu.VMEM((CHUNK,), jnp.int32),
        pltpu.VMEM((CHUNK,), jnp.int32),
        pltpu.VMEM((CHUNK,), jnp.float32),
        pltpu.VMEM((CHUNK,), jnp.float32),
        pltpu.VMEM((N,), jnp.float32),
        pltpu.VMEM((CHUNK, D), jnp.float32),
        pltpu.VMEM_SHARED((N, D), jnp.float32),
        pltpu.SemaphoreType.DMA,
    ],
)
def _agg_kernel(src_hbm, dst_hbm, ew_hbm, dinv_hbm, xw_hbm, zero_hbm, out_hbm,
                src_v, dst_v, ew_v, norm_v, dinv_v, rows, acc, sem):
    c = lax.axis_index("c")
    s = lax.axis_index("s")
    wid = s * NC + c

    pltpu.sync_copy(dinv_hbm, dinv_v)

    @pl.when(s == 0)
    def _():
        pltpu.sync_copy(zero_hbm, acc)

    plsc.subcore_barrier()

    def body(i, carry):
        base = wid * PER_W + i * CHUNK
        pltpu.sync_copy(src_hbm.at[pl.ds(base, CHUNK)], src_v)
        pltpu.sync_copy(dst_hbm.at[pl.ds(base, CHUNK)], dst_v)
        pltpu.sync_copy(ew_hbm.at[pl.ds(base, CHUNK)], ew_v)
        cp = pltpu.async_copy(xw_hbm.at[src_v], rows, sem)
        for j in range(8):
            sl = pl.ds(j * 16, 16)
            si = src_v[sl]
            di = dst_v[sl]
            nrm = (plsc.load_gather(dinv_v, [si]) * ew_v[sl]
                   * plsc.load_gather(dinv_v, [di]))
            norm_v[sl] = nrm
        cp.wait()

        def scale(e, cc):
            nv = norm_v[e]
            for j in range(8):
                fsl = pl.ds(j * 16, 16)
                rows[e, fsl] = rows[e, fsl] * nv
            return cc

        lax.fori_loop(0, CHUNK, scale, 0)
        pltpu.sync_copy(rows, acc.at[dst_v], add=True)
        return carry

    lax.fori_loop(0, NCHUNK, body, 0)
    plsc.subcore_barrier()

    @pl.when(s == 0)
    def _():
        pltpu.sync_copy(acc, out_hbm.at[c])


@functools.partial(
    pl.kernel,
    out_type=jax.ShapeDtypeStruct((NC, N * G), jnp.float32),
    mesh=_MESH,
    scratch_types=[
        pltpu.VMEM((CHUNK,), jnp.int32),
        pltpu.VMEM((CHUNK,), jnp.int32),
        pltpu.VMEM((CHUNK,), jnp.float32),
        pltpu.VMEM((CHUNK,), jnp.float32),
        pltpu.VMEM((CHUNK,), jnp.int32),
        pltpu.VMEM((N,), jnp.float32),
        pltpu.VMEM((N,), jnp.int32),
        pltpu.VMEM_SHARED((N * G,), jnp.float32),
    ],
)
def _mbuild_kernel(src_hbm, dst_hbm, ew_hbm, dinv_hbm, batch_hbm, zero_hbm,
                   out_hbm, src_v, dst_v, ew_v, val_v, fidx_v, dinv_v,
                   batch_v, acc):
    c = lax.axis_index("c")
    s = lax.axis_index("s")
    wid = s * NC + c

    pltpu.sync_copy(dinv_hbm, dinv_v)
    pltpu.sync_copy(batch_hbm, batch_v)

    @pl.when(s == 0)
    def _():
        pltpu.sync_copy(zero_hbm, acc)

    plsc.subcore_barrier()

    def body(i, carry):
        base = wid * PER_W + i * CHUNK
        pltpu.sync_copy(src_hbm.at[pl.ds(base, CHUNK)], src_v)
        pltpu.sync_copy(dst_hbm.at[pl.ds(base, CHUNK)], dst_v)
        pltpu.sync_copy(ew_hbm.at[pl.ds(base, CHUNK)], ew_v)
        for j in range(8):
            sl = pl.ds(j * 16, 16)
            si = src_v[sl]
            di = dst_v[sl]
            nrm = (plsc.load_gather(dinv_v, [si]) * ew_v[sl]
                   * plsc.load_gather(dinv_v, [di]))
            bg = plsc.load_gather(batch_v, [di])
            val_v[sl] = nrm
            fidx_v[sl] = si * G + bg
        pltpu.sync_copy(val_v, acc.at[fidx_v], add=True)
        return carry

    lax.fori_loop(0, NCHUNK, body, 0)
    plsc.subcore_barrier()

    @pl.when(s == 0)
    def _():
        pltpu.sync_copy(acc, out_hbm.at[c])


def _dinv_body(d0_ref, d1_ref, o_ref):
    deg = d0_ref[...] + d1_ref[...]
    safe = jnp.where(deg > 0.0, deg, 1.0)
    o_ref[...] = jnp.where(deg > 0.0, lax.rsqrt(safe), 0.0)


def _mm_body(x_ref, w_ref, o_ref):
    o_ref[...] = jnp.dot(x_ref[...], w_ref[...],
                         preferred_element_type=jnp.float32)


ROWB = 400
NBLK = N // ROWB


def _final_body(p0_ref, p1_ref, b1_ref, m0_ref, m1_ref, bt_ref, w2_ref,
                b2_ref, wl1_ref, bl1_ref, wl2_ref, bl2_ref, o_ref,
                gacc, cnt):
    i = pl.program_id(0)

    @pl.when(i == 0)
    def _():
        gacc[...] = jnp.zeros_like(gacc)
        cnt[...] = jnp.zeros_like(cnt)

    h = jnp.maximum(p0_ref[...] + p1_ref[...] + b1_ref[...], 0.0)
    m = m0_ref[...] + m1_ref[...]
    gacc[...] += lax.dot_general(m, h, (((0,), (0,)), ((), ())),
                                 preferred_element_type=jnp.float32)
    bt = bt_ref[...].reshape(1, ROWB)
    onehot = (lax.broadcasted_iota(jnp.int32, (G, ROWB), 0) == bt)
    cnt[...] += jnp.sum(onehot.astype(jnp.float32), axis=1, keepdims=True)

    @pl.when(i == NBLK - 1)
    def _():
        cc = cnt[...]
        g64 = gacc[...] / jnp.maximum(cc, 1.0)
        mask = jnp.where(cc > 0.0, 1.0, 0.0)
        gp = (jnp.dot(g64, w2_ref[...], preferred_element_type=jnp.float32)
              + mask * b2_ref[...])
        z = (jnp.dot(gp, wl1_ref[...], preferred_element_type=jnp.float32)
             + bl1_ref[...])
        o_ref[...] = (jnp.dot(z, wl2_ref[...],
                              preferred_element_type=jnp.float32)
                      + bl2_ref[...])


def kernel(x, edge_index, edge_attr, batch, W1, b1, W2, b2, Wl1, bl1, Wl2,
           bl2):
    f32 = jnp.float32
    src = edge_index[0]
    dst = edge_index[1]
    loop = jnp.arange(N, dtype=jnp.int32)
    pad = EP - E - N
    src2 = jnp.concatenate([src, loop, jnp.zeros((pad,), jnp.int32)])
    dst2 = jnp.concatenate([dst, loop, jnp.zeros((pad,), jnp.int32)])
    ew2 = jnp.concatenate([edge_attr, jnp.ones((N,), f32),
                           jnp.zeros((pad,), f32)])

    z1 = jnp.zeros((N,), f32)
    zrow = jnp.zeros((N, D), f32)
    zm = jnp.zeros((N * G,), f32)

    degp = _deg_kernel(dst2, ew2, z1)

    dinv2d = pl.pallas_call(
        _dinv_body,
        out_shape=jax.ShapeDtypeStruct((80, 125), f32),
    )(degp[0].reshape(80, 125), degp[1].reshape(80, 125))
    dinv = dinv2d.reshape(N)

    xw = pl.pallas_call(
        _mm_body,
        grid=(NBLK,),
        in_specs=[
            pl.BlockSpec((ROWB, D), lambda i: (i, 0)),
            pl.BlockSpec((D, D), lambda i: (0, 0)),
        ],
        out_specs=pl.BlockSpec((ROWB, D), lambda i: (i, 0)),
        out_shape=jax.ShapeDtypeStruct((N, D), f32),
    )(x, W1)

    aggp = _agg_kernel(src2, dst2, ew2, dinv, xw, zrow)
    mrawp = _mbuild_kernel(src2, dst2, ew2, dinv, batch, zm)
    m0 = mrawp[0].reshape(N, G)
    m1 = mrawp[1].reshape(N, G)
    bt3 = batch.reshape(NBLK, 1, ROWB)

    out = pl.pallas_call(
        _final_body,
        grid=(NBLK,),
        in_specs=[
            pl.BlockSpec((ROWB, D), lambda i: (i, 0)),
            pl.BlockSpec((ROWB, D), lambda i: (i, 0)),
            pl.BlockSpec((1, D), lambda i: (0, 0)),
            pl.BlockSpec((ROWB, G), lambda i: (i, 0)),
            pl.BlockSpec((ROWB, G), lambda i: (i, 0)),
            pl.BlockSpec((1, 1, ROWB), lambda i: (i, 0, 0)),
            pl.BlockSpec((D, D), lambda i: (0, 0)),
            pl.BlockSpec((1, D), lambda i: (0, 0)),
            pl.BlockSpec((D, 32), lambda i: (0, 0)),
            pl.BlockSpec((1, 32), lambda i: (0, 0)),
            pl.BlockSpec((32, 10), lambda i: (0, 0)),
            pl.BlockSpec((1, 10), lambda i: (0, 0)),
        ],
        out_specs=pl.BlockSpec((G, 10), lambda i: (0, 0)),
        out_shape=jax.ShapeDtypeStruct((G, 10), f32),
        scratch_shapes=[
            pltpu.VMEM((G, D), f32),
            pltpu.VMEM((G, 1), f32),
        ],
    )(aggp[0], aggp[1], b1.reshape(1, D), m0, m1, bt3, W2,
      b2.reshape(1, D), Wl1, bl1.reshape(1, 32), Wl2, bl2.reshape(1, 10))
    return out


# speedup vs baseline: 10.4639x; 10.4639x over previous
import functools

import jax
import jax.numpy as jnp
from jax import lax
from jax.experimental import pallas as pl
from jax.experimental.pallas import tpu as pltpu
from jax.experimental.pallas import tpu_sc as plsc

N = 10000
E = 320000
D = 128
G = 64
NC = 2
NS = 16
NW = NC * NS
CHUNK = 128
EP = 331776
NCHUNK = EP // (NW * CHUNK)
PER_W = EP // NW

_MESH = plsc.VectorSubcoreMesh(core_axis_name="c", subcore_axis_name="s")


@functools.partial(
    pl.kernel,
    out_type=jax.ShapeDtypeStruct((NC, N), jnp.float32),
    mesh=_MESH,
    scratch_types=[
        pltpu.VMEM((CHUNK,), jnp.int32),
        pltpu.VMEM((CHUNK,), jnp.float32),
        pltpu.VMEM_SHARED((N,), jnp.float32),
    ],
)
def _deg_kernel(dst_hbm, ew_hbm, zero_hbm, out_hbm, idx_v, val_v, acc):
    c = lax.axis_index("c")
    s = lax.axis_index("s")
    wid = s * NC + c

    @pl.when(s == 0)
    def _():
        pltpu.sync_copy(zero_hbm, acc)

    plsc.subcore_barrier()

    def body(i, carry):
        base = wid * PER_W + i * CHUNK
        pltpu.sync_copy(dst_hbm.at[pl.ds(base, CHUNK)], idx_v)
        pltpu.sync_copy(ew_hbm.at[pl.ds(base, CHUNK)], val_v)
        pltpu.sync_copy(val_v, acc.at[idx_v], add=True)
        return carry

    lax.fori_loop(0, NCHUNK, body, 0)
    plsc.subcore_barrier()

    @pl.when(s == 0)
    def _():
        pltpu.sync_copy(acc, out_hbm.at[c])


@functools.partial(
    pl.kernel,
    out_type=jax.ShapeDtypeStruct((NC, N, D), jnp.float32),
    mesh=_MESH,
    scratch_types=[
        pltpu.VMEM((CHUNK,), jnp.int32),
        pltpu.VMEM((CHUNK,), jnp.int32),
        pltpu.VMEM((CHUNK,), jnp.float32),
        pltpu.VMEM((CHUNK,), jnp.float32),
        pltpu.VMEM((CHUNK,), jnp.float32),
        pltpu.VMEM((CHUNK,), jnp.float32),
        pltpu.VMEM((CHUNK, D), jnp.float32),
        pltpu.VMEM_SHARED((N, D), jnp.float32),
        pltpu.SemaphoreType.DMA,
        pltpu.SemaphoreType.DMA,
        pltpu.SemaphoreType.DMA,
    ],
)
def _agg_kernel(src_hbm, dst_hbm, ew_hbm, dinv_hbm, xw_hbm, zero_hbm, out_hbm,
                src_v, dst_v, ew_v, norm_v, dsrc_v, ddst_v, rows, acc,
                sem, sem2, sem3):
    c = lax.axis_index("c")
    s = lax.axis_index("s")
    wid = s * NC + c

    @pl.when(s == 0)
    def _():
        pltpu.sync_copy(zero_hbm, acc)

    plsc.subcore_barrier()

    def body(i, carry):
        base = wid * PER_W + i * CHUNK
        pltpu.sync_copy(src_hbm.at[pl.ds(base, CHUNK)], src_v)
        pltpu.sync_copy(dst_hbm.at[pl.ds(base, CHUNK)], dst_v)
        pltpu.sync_copy(ew_hbm.at[pl.ds(base, CHUNK)], ew_v)
        cp = pltpu.async_copy(xw_hbm.at[src_v], rows, sem)
        cps = pltpu.async_copy(dinv_hbm.at[src_v], dsrc_v, sem2)
        cpd = pltpu.async_copy(dinv_hbm.at[dst_v], ddst_v, sem3)
        cps.wait()
        cpd.wait()
        for j in range(8):
            sl = pl.ds(j * 16, 16)
            norm_v[sl] = dsrc_v[sl] * ew_v[sl] * ddst_v[sl]
        cp.wait()

        def scale16(g, cc):
            n16 = norm_v[pl.ds(g * 16, 16)]
            for l in range(16):
                nv = n16[l]
                e = g * 16 + l
                for j in range(8):
                    fsl = pl.ds(j * 16, 16)
                    rows[e, fsl] = rows[e, fsl] * nv
            return cc

        lax.fori_loop(0, CHUNK // 16, scale16, 0)
        pltpu.sync_copy(rows, acc.at[dst_v], add=True)
        return carry

    lax.fori_loop(0, NCHUNK, body, 0)
    plsc.subcore_barrier()

    @pl.when(s == 0)
    def _():
        pltpu.sync_copy(acc, out_hbm.at[c])


@functools.partial(
    pl.kernel,
    out_type=jax.ShapeDtypeStruct((NC, N * G), jnp.float32),
    mesh=_MESH,
    scratch_types=[
        pltpu.VMEM((CHUNK,), jnp.int32),
        pltpu.VMEM((CHUNK,), jnp.int32),
        pltpu.VMEM((CHUNK,), jnp.float32),
        pltpu.VMEM((CHUNK,), jnp.float32),
        pltpu.VMEM((CHUNK,), jnp.int32),
        pltpu.VMEM((CHUNK,), jnp.float32),
        pltpu.VMEM((CHUNK,), jnp.float32),
        pltpu.VMEM((CHUNK,), jnp.int32),
        pltpu.VMEM_SHARED((N * G,), jnp.float32),
        pltpu.SemaphoreType.DMA,
        pltpu.SemaphoreType.DMA,
        pltpu.SemaphoreType.DMA,
    ],
)
def _mbuild_kernel(src_hbm, dst_hbm, ew_hbm, dinv_hbm, batch_hbm, zero_hbm,
                   out_hbm, src_v, dst_v, ew_v, val_v, fidx_v, dsrc_v,
                   ddst_v, bd_v, acc, sem, sem2, sem3):
    c = lax.axis_index("c")
    s = lax.axis_index("s")
    wid = s * NC + c

    @pl.when(s == 0)
    def _():
        pltpu.sync_copy(zero_hbm, acc)

    plsc.subcore_barrier()

    def body(i, carry):
        base = wid * PER_W + i * CHUNK
        pltpu.sync_copy(src_hbm.at[pl.ds(base, CHUNK)], src_v)
        pltpu.sync_copy(dst_hbm.at[pl.ds(base, CHUNK)], dst_v)
        pltpu.sync_copy(ew_hbm.at[pl.ds(base, CHUNK)], ew_v)
        cps = pltpu.async_copy(dinv_hbm.at[src_v], dsrc_v, sem)
        cpd = pltpu.async_copy(dinv_hbm.at[dst_v], ddst_v, sem2)
        cpb = pltpu.async_copy(batch_hbm.at[dst_v], bd_v, sem3)
        cps.wait()
        cpd.wait()
        cpb.wait()
        for j in range(8):
            sl = pl.ds(j * 16, 16)
            val_v[sl] = dsrc_v[sl] * ew_v[sl] * ddst_v[sl]
            fidx_v[sl] = src_v[sl] * G + bd_v[sl]
        pltpu.sync_copy(val_v, acc.at[fidx_v], add=True)
        return carry

    lax.fori_loop(0, NCHUNK, body, 0)
    plsc.subcore_barrier()

    @pl.when(s == 0)
    def _():
        pltpu.sync_copy(acc, out_hbm.at[c])


def _dinv_body(d0_ref, d1_ref, o_ref):
    deg = d0_ref[...] + d1_ref[...]
    safe = jnp.where(deg > 0.0, deg, 1.0)
    o_ref[...] = jnp.where(deg > 0.0, lax.rsqrt(safe), 0.0)


def _mm_body(x_ref, w_ref, o_ref):
    o_ref[...] = jnp.dot(x_ref[...], w_ref[...],
                         preferred_element_type=jnp.float32)


ROWB = 400
NBLK = N // ROWB


def _final_body(p0_ref, p1_ref, b1_ref, m0_ref, m1_ref, bt_ref, w2_ref,
                b2_ref, wl1_ref, bl1_ref, wl2_ref, bl2_ref, o_ref,
                gacc, cnt):
    i = pl.program_id(0)

    @pl.when(i == 0)
    def _():
        gacc[...] = jnp.zeros_like(gacc)
        cnt[...] = jnp.zeros_like(cnt)

    h = jnp.maximum(p0_ref[...] + p1_ref[...] + b1_ref[...], 0.0)
    m = m0_ref[...] + m1_ref[...]
    gacc[...] += lax.dot_general(m, h, (((0,), (0,)), ((), ())),
                                 preferred_element_type=jnp.float32)
    bt = bt_ref[...].reshape(1, ROWB)
    onehot = (lax.broadcasted_iota(jnp.int32, (G, ROWB), 0) == bt)
    cnt[...] += jnp.sum(onehot.astype(jnp.float32), axis=1, keepdims=True)

    @pl.when(i == NBLK - 1)
    def _():
        cc = cnt[...]
        g64 = gacc[...] / jnp.maximum(cc, 1.0)
        mask = jnp.where(cc > 0.0, 1.0, 0.0)
        gp = (jnp.dot(g64, w2_ref[...], preferred_element_type=jnp.float32)
              + mask * b2_ref[...])
        z = (jnp.dot(gp, wl1_ref[...], preferred_element_type=jnp.float32)
             + bl1_ref[...])
        o_ref[...] = (jnp.dot(z, wl2_ref[...],
                              preferred_element_type=jnp.float32)
                      + bl2_ref[...])


def kernel(x, edge_index, edge_attr, batch, W1, b1, W2, b2, Wl1, bl1, Wl2,
           bl2):
    f32 = jnp.float32
    src = edge_index[0]
    dst = edge_index[1]
    loop = jnp.arange(N, dtype=jnp.int32)
    pad = EP - E - N
    src2 = jnp.concatenate([src, loop, jnp.zeros((pad,), jnp.int32)])
    dst2 = jnp.concatenate([dst, loop, jnp.zeros((pad,), jnp.int32)])
    ew2 = jnp.concatenate([edge_attr, jnp.ones((N,), f32),
                           jnp.zeros((pad,), f32)])

    z1 = jnp.zeros((N,), f32)
    zrow = jnp.zeros((N, D), f32)
    zm = jnp.zeros((N * G,), f32)

    degp = _deg_kernel(dst2, ew2, z1)

    dinv2d = pl.pallas_call(
        _dinv_body,
        out_shape=jax.ShapeDtypeStruct((80, 125), f32),
    )(degp[0].reshape(80, 125), degp[1].reshape(80, 125))
    dinv = dinv2d.reshape(N)

    xw = pl.pallas_call(
        _mm_body,
        grid=(NBLK,),
        in_specs=[
            pl.BlockSpec((ROWB, D), lambda i: (i, 0)),
            pl.BlockSpec((D, D), lambda i: (0, 0)),
        ],
        out_specs=pl.BlockSpec((ROWB, D), lambda i: (i, 0)),
        out_shape=jax.ShapeDtypeStruct((N, D), f32),
    )(x, W1)

    aggp = _agg_kernel(src2, dst2, ew2, dinv, xw, zrow)
    mrawp = _mbuild_kernel(src2, dst2, ew2, dinv, batch, zm)
    m0 = mrawp[0].reshape(N, G)
    m1 = mrawp[1].reshape(N, G)
    bt3 = batch.reshape(NBLK, 1, ROWB)

    out = pl.pallas_call(
        _final_body,
        grid=(NBLK,),
        in_specs=[
            pl.BlockSpec((ROWB, D), lambda i: (i, 0)),
            pl.BlockSpec((ROWB, D), lambda i: (i, 0)),
            pl.BlockSpec((1, D), lambda i: (0, 0)),
            pl.BlockSpec((ROWB, G), lambda i: (i, 0)),
            pl.BlockSpec((ROWB, G), lambda i: (i, 0)),
            pl.BlockSpec((1, 1, ROWB), lambda i: (i, 0, 0)),
            pl.BlockSpec((D, D), lambda i: (0, 0)),
            pl.BlockSpec((1, D), lambda i: (0, 0)),
            pl.BlockSpec((D, 32), lambda i: (0, 0)),
            pl.BlockSpec((1, 32), lambda i: (0, 0)),
            pl.BlockSpec((32, 10), lambda i: (0, 0)),
            pl.BlockSpec((1, 10), lambda i: (0, 0)),
        ],
        out_specs=pl.BlockSpec((G, 10), lambda i: (0, 0)),
        out_shape=jax.ShapeDtypeStruct((G, 10), f32),
        scratch_shapes=[
            pltpu.VMEM((G, D), f32),
            pltpu.VMEM((G, 1), f32),
        ],
    )(aggp[0], aggp[1], b1.reshape(1, D), m0, m1, bt3, W2,
      b2.reshape(1, D), Wl1, bl1.reshape(1, 32), Wl2, bl2.reshape(1, 10))
    return out


# speedup vs baseline: 10.8670x; 1.0385x over previous
import functools

import jax
import jax.numpy as jnp
from jax import lax
from jax.experimental import pallas as pl
from jax.experimental.pallas import tpu as pltpu
from jax.experimental.pallas import tpu_sc as plsc

N = 10000
E = 320000
D = 128
G = 64
NC = 2
NS = 16
NW = NC * NS
CHUNK = 128
WCHUNK = 84
PER_W = WCHUNK * CHUNK
EP = NW * PER_W

_MESH = plsc.VectorSubcoreMesh(core_axis_name="c", subcore_axis_name="s")


@functools.partial(
    pl.kernel,
    out_type=jax.ShapeDtypeStruct((NC, N), jnp.float32),
    mesh=_MESH,
    scratch_types=[
        pltpu.VMEM((WCHUNK, CHUNK), jnp.int32),
        pltpu.VMEM((WCHUNK, CHUNK), jnp.float32),
        pltpu.VMEM_SHARED((N,), jnp.float32),
        pltpu.SemaphoreType.DMA,
    ],
)
def _deg_kernel(dst_hbm, ew_hbm, zero_hbm, out_hbm, dst_m, ew_m, acc, sem):
    c = lax.axis_index("c")
    s = lax.axis_index("s")
    wid = s * NC + c

    pltpu.sync_copy(dst_hbm.at[wid], dst_m)
    pltpu.sync_copy(ew_hbm.at[wid], ew_m)

    @pl.when(s == 0)
    def _():
        pltpu.sync_copy(zero_hbm, acc)

    plsc.subcore_barrier()

    def fire(i, carry):
        pltpu.async_copy(ew_m.at[i], acc.at[dst_m.at[i]], sem, add=True)
        return carry

    lax.fori_loop(0, WCHUNK, fire, 0)

    def drain(i, carry):
        pltpu.make_async_copy(ew_m.at[0], acc.at[dst_m.at[0]], sem).wait()
        return carry

    lax.fori_loop(0, WCHUNK, drain, 0)
    plsc.subcore_barrier()

    @pl.when(s == 0)
    def _():
        pltpu.sync_copy(acc, out_hbm.at[c])


@functools.partial(
    pl.kernel,
    out_type=(
        jax.ShapeDtypeStruct((NC, N, D), jnp.float32),
        jax.ShapeDtypeStruct((EP,), jnp.float32),
        jax.ShapeDtypeStruct((EP,), jnp.int32),
    ),
    mesh=_MESH,
    scratch_types=[
        pltpu.VMEM((PER_W,), jnp.int32),
        [pltpu.VMEM((CHUNK,), jnp.int32) for _ in range(3)],
        [pltpu.VMEM((CHUNK,), jnp.float32) for _ in range(3)],
        [pltpu.VMEM((CHUNK, D), jnp.float32) for _ in range(2)],
        [pltpu.VMEM((CHUNK,), jnp.float32) for _ in range(2)],
        [pltpu.VMEM((CHUNK,), jnp.float32) for _ in range(2)],
        [pltpu.VMEM((CHUNK,), jnp.int32) for _ in range(2)],
        [pltpu.VMEM((CHUNK,), jnp.float32) for _ in range(2)],
        [pltpu.VMEM((CHUNK,), jnp.int32) for _ in range(2)],
        pltpu.VMEM_SHARED((N, D), jnp.float32),
        [pltpu.SemaphoreType.DMA for _ in range(3)],
        [pltpu.SemaphoreType.DMA for _ in range(2)],
        [pltpu.SemaphoreType.DMA for _ in range(2)],
    ],
)
def _agg_kernel(src_hbm, dst_hbm, ew_hbm, dinv_hbm, batch_hbm, xw_hbm,
                zrow_hbm, outa_hbm, outn_hbm, outf_hbm,
                srcb, dstb, ewb, rowsb, dsrcb, ddstb, bdb, normb, fidxb,
                acca, semL, semG, semW):
    c = lax.axis_index("c")
    s = lax.axis_index("s")
    wid = s * NC + c
    base_w = wid * PER_W

    pltpu.sync_copy(src_hbm.at[pl.ds(base_w, PER_W)], srcb)

    @pl.when(s == 0)
    def _():
        pltpu.sync_copy(zrow_hbm, acca)

    def issue_l(i, kl):
        off = base_w + i * CHUNK
        pltpu.async_copy(dst_hbm.at[pl.ds(off, CHUNK)], dstb[kl], semL[kl])
        pltpu.async_copy(ew_hbm.at[pl.ds(off, CHUNK)], ewb[kl], semL[kl])

    def wait_l(kl):
        pltpu.make_async_copy(dst_hbm.at[pl.ds(0, CHUNK)], dstb[kl],
                              semL[kl]).wait()
        pltpu.make_async_copy(ew_hbm.at[pl.ds(0, CHUNK)], ewb[kl],
                              semL[kl]).wait()

    def issue_g(i, kg, kl):
        sidx = srcb.at[pl.ds(i * CHUNK, CHUNK)]
        pltpu.async_copy(xw_hbm.at[sidx], rowsb[kg], semG[kg])
        pltpu.async_copy(dinv_hbm.at[sidx], dsrcb[kg], semG[kg])
        pltpu.async_copy(dinv_hbm.at[dstb[kl]], ddstb[kg], semG[kg])
        pltpu.async_copy(batch_hbm.at[dstb[kl]], bdb[kg], semG[kg])

    def wait_g(kg):
        sidx0 = srcb.at[pl.ds(0, CHUNK)]
        pltpu.make_async_copy(xw_hbm.at[sidx0], rowsb[kg], semG[kg]).wait()
        pltpu.make_async_copy(dinv_hbm.at[sidx0], dsrcb[kg], semG[kg]).wait()
        pltpu.make_async_copy(dinv_hbm.at[sidx0], ddstb[kg], semG[kg]).wait()
        pltpu.make_async_copy(batch_hbm.at[sidx0], bdb[kg], semG[kg]).wait()

    def wait_w(kg):
        off0 = pl.ds(0, CHUNK)
        pltpu.make_async_copy(normb[kg], outn_hbm.at[off0], semW[kg]).wait()
        pltpu.make_async_copy(fidxb[kg], outf_hbm.at[off0], semW[kg]).wait()

    def process(i, kg, kl):
        rows = rowsb[kg]
        norm = normb[kg]
        fidx = fidxb[kg]

        @pl.when(i >= 2)
        def _():
            wait_w(kg)

        for j in range(8):
            sl = pl.ds(j * 16, 16)
            ssl = pl.ds(i * CHUNK + j * 16, 16)
            norm[sl] = dsrcb[kg][sl] * ewb[kl][sl] * ddstb[kg][sl]
            fidx[sl] = srcb[ssl] * G + bdb[kg][sl]

        def scale16(g, cc):
            n16 = norm[pl.ds(g * 16, 16)]
            for l in range(16):
                nv = n16[l]
                e = g * 16 + l
                for j in range(8):
                    fsl = pl.ds(j * 16, 16)
                    rows[e, fsl] = rows[e, fsl] * nv
            return cc

        lax.fori_loop(0, CHUNK // 16, scale16, 0)
        pltpu.sync_copy(rows, acca.at[dstb[kl]], add=True)
        off = base_w + i * CHUNK
        pltpu.async_copy(norm, outn_hbm.at[pl.ds(off, CHUNK)], semW[kg])
        pltpu.async_copy(fidx, outf_hbm.at[pl.ds(off, CHUNK)], semW[kg])

    plsc.subcore_barrier()

    issue_l(0, 0)
    issue_l(1, 1)
    issue_l(2, 2)
    wait_l(0)
    issue_g(0, 0, 0)

    def outer(p, carry):
        for k in range(6):
            i = 6 * p + k
            kg = k % 2
            kl = k % 3
            kg1 = (k + 1) % 2
            kl1 = (k + 1) % 3

            @pl.when(i + 1 < WCHUNK)
            def _():
                wait_l(kl1)
                issue_g(i + 1, kg1, kl1)

            wait_g(kg)
            process(i, kg, kl)

            @pl.when(i + 3 < WCHUNK)
            def _():
                issue_l(i + 3, kl)

        return carry

    lax.fori_loop(0, WCHUNK // 6, outer, 0)
    wait_w(0)
    wait_w(1)
    plsc.subcore_barrier()

    @pl.when(s == 0)
    def _():
        pltpu.sync_copy(acca, outa_hbm.at[c])


@functools.partial(
    pl.kernel,
    out_type=jax.ShapeDtypeStruct((NC, N * G), jnp.float32),
    mesh=_MESH,
    scratch_types=[
        pltpu.VMEM((WCHUNK, CHUNK), jnp.float32),
        pltpu.VMEM((WCHUNK, CHUNK), jnp.int32),
        pltpu.VMEM_SHARED((N * G,), jnp.float32),
        pltpu.SemaphoreType.DMA,
    ],
)
def _mbuild_kernel(norm_hbm, fidx_hbm, zero_hbm, out_hbm, norm_m, fidx_m,
                   acc, sem):
    c = lax.axis_index("c")
    s = lax.axis_index("s")
    wid = s * NC + c

    pltpu.sync_copy(norm_hbm.at[wid], norm_m)
    pltpu.sync_copy(fidx_hbm.at[wid], fidx_m)

    @pl.when(s == 0)
    def _():
        pltpu.sync_copy(zero_hbm, acc)

    plsc.subcore_barrier()

    def fire(i, carry):
        pltpu.async_copy(norm_m.at[i], acc.at[fidx_m.at[i]], sem, add=True)
        return carry

    lax.fori_loop(0, WCHUNK, fire, 0)

    def drain(i, carry):
        pltpu.make_async_copy(norm_m.at[0], acc.at[fidx_m.at[0]], sem).wait()
        return carry

    lax.fori_loop(0, WCHUNK, drain, 0)
    plsc.subcore_barrier()

    @pl.when(s == 0)
    def _():
        pltpu.sync_copy(acc, out_hbm.at[c])


def _dinv_body(d0_ref, d1_ref, o_ref):
    deg = d0_ref[...] + d1_ref[...]
    safe = jnp.where(deg > 0.0, deg, 1.0)
    o_ref[...] = jnp.where(deg > 0.0, lax.rsqrt(safe), 0.0)


def _mm_body(x_ref, w_ref, o_ref):
    o_ref[...] = jnp.dot(x_ref[...], w_ref[...],
                         preferred_element_type=jnp.float32)


ROWB = 400
NBLK = N // ROWB


def _final_body(p0_ref, p1_ref, b1_ref, m0_ref, m1_ref, bt_ref, w2_ref,
                b2_ref, wl1_ref, bl1_ref, wl2_ref, bl2_ref, o_ref,
                gacc, cnt):
    i = pl.program_id(0)

    @pl.when(i == 0)
    def _():
        gacc[...] = jnp.zeros_like(gacc)
        cnt[...] = jnp.zeros_like(cnt)

    h = jnp.maximum(p0_ref[...] + p1_ref[...] + b1_ref[...], 0.0)
    m = m0_ref[...] + m1_ref[...]
    gacc[...] += lax.dot_general(m, h, (((0,), (0,)), ((), ())),
                                 preferred_element_type=jnp.float32)
    bt = bt_ref[...].reshape(1, ROWB)
    onehot = (lax.broadcasted_iota(jnp.int32, (G, ROWB), 0) == bt)
    cnt[...] += jnp.sum(onehot.astype(jnp.float32), axis=1, keepdims=True)

    @pl.when(i == NBLK - 1)
    def _():
        cc = cnt[...]
        g64 = gacc[...] / jnp.maximum(cc, 1.0)
        mask = jnp.where(cc > 0.0, 1.0, 0.0)
        gp = (jnp.dot(g64, w2_ref[...], preferred_element_type=jnp.float32)
              + mask * b2_ref[...])
        z = (jnp.dot(gp, wl1_ref[...], preferred_element_type=jnp.float32)
             + bl1_ref[...])
        o_ref[...] = (jnp.dot(z, wl2_ref[...],
                              preferred_element_type=jnp.float32)
                      + bl2_ref[...])


def kernel(x, edge_index, edge_attr, batch, W1, b1, W2, b2, Wl1, bl1, Wl2,
           bl2):
    f32 = jnp.float32
    src = edge_index[0]
    dst = edge_index[1]
    loop = jnp.arange(N, dtype=jnp.int32)
    pad = EP - E - N
    src1 = jnp.concatenate([src, loop, jnp.zeros((pad,), jnp.int32)])
    dst1 = jnp.concatenate([dst, loop, jnp.zeros((pad,), jnp.int32)])
    ew1 = jnp.concatenate([edge_attr, jnp.ones((N,), f32),
                           jnp.zeros((pad,), f32)])
    dst3 = dst1.reshape(NW, WCHUNK, CHUNK)
    ew3 = ew1.reshape(NW, WCHUNK, CHUNK)

    z1 = jnp.zeros((N,), f32)
    zrow = jnp.zeros((N, D), f32)
    zm = jnp.zeros((N * G,), f32)

    degp = _deg_kernel(dst3, ew3, z1)

    dinv2d = pl.pallas_call(
        _dinv_body,
        out_shape=jax.ShapeDtypeStruct((80, 125), f32),
    )(degp[0].reshape(80, 125), degp[1].reshape(80, 125))
    dinv = dinv2d.reshape(N)

    xw = pl.pallas_call(
        _mm_body,
        grid=(NBLK,),
        in_specs=[
            pl.BlockSpec((ROWB, D), lambda i: (i, 0)),
            pl.BlockSpec((D, D), lambda i: (0, 0)),
        ],
        out_specs=pl.BlockSpec((ROWB, D), lambda i: (i, 0)),
        out_shape=jax.ShapeDtypeStruct((N, D), f32),
    )(x, W1)

    aggp, normf, fidxf = _agg_kernel(src1, dst1, ew1, dinv, batch, xw, zrow)
    mrawp = _mbuild_kernel(normf.reshape(NW, WCHUNK, CHUNK),
                           fidxf.reshape(NW, WCHUNK, CHUNK), zm)
    m0 = mrawp[0].reshape(N, G)
    m1 = mrawp[1].reshape(N, G)
    bt3 = batch.reshape(NBLK, 1, ROWB)

    out = pl.pallas_call(
        _final_body,
        grid=(NBLK,),
        in_specs=[
            pl.BlockSpec((ROWB, D), lambda i: (i, 0)),
            pl.BlockSpec((ROWB, D), lambda i: (i, 0)),
            pl.BlockSpec((1, D), lambda i: (0, 0)),
            pl.BlockSpec((ROWB, G), lambda i: (i, 0)),
            pl.BlockSpec((ROWB, G), lambda i: (i, 0)),
            pl.BlockSpec((1, 1, ROWB), lambda i: (i, 0, 0)),
            pl.BlockSpec((D, D), lambda i: (0, 0)),
            pl.BlockSpec((1, D), lambda i: (0, 0)),
            pl.BlockSpec((D, 32), lambda i: (0, 0)),
            pl.BlockSpec((1, 32), lambda i: (0, 0)),
            pl.BlockSpec((32, 10), lambda i: (0, 0)),
            pl.BlockSpec((1, 10), lambda i: (0, 0)),
        ],
        out_specs=pl.BlockSpec((G, 10), lambda i: (0, 0)),
        out_shape=jax.ShapeDtypeStruct((G, 10), f32),
        scratch_shapes=[
            pltpu.VMEM((G, D), f32),
            pltpu.VMEM((G, 1), f32),
        ],
    )(aggp[0], aggp[1], b1.reshape(1, D), m0, m1, bt3, W2,
      b2.reshape(1, D), Wl1, bl1.reshape(1, 32), Wl2, bl2.reshape(1, 10))
    return out


# speedup vs baseline: 11.3540x; 1.0448x over previous
import functools

import jax
import jax.numpy as jnp
from jax import lax
from jax.experimental import pallas as pl
from jax.experimental.pallas import tpu as pltpu
from jax.experimental.pallas import tpu_sc as plsc

N = 10000
E = 320000
D = 128
G = 64
NC = 2
NS = 16
NW = NC * NS
CHUNK = 112
WCHUNK = 96
PER_W = WCHUNK * CHUNK
EP = NW * PER_W

_MESH = plsc.VectorSubcoreMesh(core_axis_name="c", subcore_axis_name="s")


@functools.partial(
    pl.kernel,
    out_type=jax.ShapeDtypeStruct((NC, N), jnp.float32),
    mesh=_MESH,
    scratch_types=[
        pltpu.VMEM((WCHUNK, CHUNK), jnp.int32),
        pltpu.VMEM((WCHUNK, CHUNK), jnp.float32),
        pltpu.VMEM_SHARED((N,), jnp.float32),
        pltpu.SemaphoreType.DMA,
    ],
)
def _deg_kernel(dst_hbm, ew_hbm, zero_hbm, out_hbm, dst_m, ew_m, acc, sem):
    c = lax.axis_index("c")
    s = lax.axis_index("s")
    wid = s * NC + c

    pltpu.sync_copy(dst_hbm.at[wid], dst_m)
    pltpu.sync_copy(ew_hbm.at[wid], ew_m)

    @pl.when(s == 0)
    def _():
        pltpu.sync_copy(zero_hbm, acc)

    plsc.subcore_barrier()

    def fire(i, carry):
        pltpu.async_copy(ew_m.at[i], acc.at[dst_m.at[i]], sem, add=True)
        return carry

    lax.fori_loop(0, WCHUNK, fire, 0)

    def drain(i, carry):
        pltpu.make_async_copy(ew_m.at[0], acc.at[dst_m.at[0]], sem).wait()
        return carry

    lax.fori_loop(0, WCHUNK, drain, 0)
    plsc.subcore_barrier()

    @pl.when(s == 0)
    def _():
        pltpu.sync_copy(acc, out_hbm.at[c])


@functools.partial(
    pl.kernel,
    out_type=(
        jax.ShapeDtypeStruct((NC, N, D), jnp.float32),
        jax.ShapeDtypeStruct((EP,), jnp.float32),
        jax.ShapeDtypeStruct((EP,), jnp.int32),
    ),
    mesh=_MESH,
    scratch_types=[
        [pltpu.VMEM((CHUNK,), jnp.int32) for _ in range(4)],
        [pltpu.VMEM((CHUNK,), jnp.int32) for _ in range(4)],
        [pltpu.VMEM((CHUNK,), jnp.float32) for _ in range(4)],
        [pltpu.VMEM((CHUNK, D), jnp.float32) for _ in range(3)],
        [pltpu.VMEM((CHUNK,), jnp.float32) for _ in range(3)],
        [pltpu.VMEM((CHUNK,), jnp.float32) for _ in range(3)],
        [pltpu.VMEM((CHUNK,), jnp.int32) for _ in range(3)],
        [pltpu.VMEM((CHUNK,), jnp.float32) for _ in range(2)],
        [pltpu.VMEM((CHUNK,), jnp.int32) for _ in range(2)],
        pltpu.VMEM_SHARED((N, D), jnp.float32),
        [pltpu.SemaphoreType.DMA for _ in range(4)],
        [pltpu.SemaphoreType.DMA for _ in range(3)],
        [pltpu.SemaphoreType.DMA for _ in range(3)],
        [pltpu.SemaphoreType.DMA for _ in range(2)],
    ],
)
def _agg_kernel(src_hbm, dst_hbm, ew_hbm, dinv_hbm, batch_hbm, xw_hbm,
                zrow_hbm, outa_hbm, outn_hbm, outf_hbm,
                srcb, dstb, ewb, rowsb, dsrcb, ddstb, bdb, normb, fidxb,
                acca, semL, semG, semS, semW):
    c = lax.axis_index("c")
    s = lax.axis_index("s")
    wid = s * NC + c
    base_w = wid * PER_W

    @pl.when(s == 0)
    def _():
        pltpu.sync_copy(zrow_hbm, acca)

    def issue_l(i, kl):
        off = base_w + i * CHUNK
        pltpu.async_copy(src_hbm.at[pl.ds(off, CHUNK)], srcb[kl], semL[kl])
        pltpu.async_copy(dst_hbm.at[pl.ds(off, CHUNK)], dstb[kl], semL[kl])
        pltpu.async_copy(ew_hbm.at[pl.ds(off, CHUNK)], ewb[kl], semL[kl])

    def wait_l(kl):
        off0 = pl.ds(0, CHUNK)
        pltpu.make_async_copy(src_hbm.at[off0], srcb[kl], semL[kl]).wait()
        pltpu.make_async_copy(dst_hbm.at[off0], dstb[kl], semL[kl]).wait()
        pltpu.make_async_copy(ew_hbm.at[off0], ewb[kl], semL[kl]).wait()

    def issue_g(i, kg, kl):
        pltpu.async_copy(xw_hbm.at[srcb[kl]], rowsb[kg], semG[kg])
        pltpu.async_copy(dinv_hbm.at[srcb[kl]], dsrcb[kg], semG[kg])
        pltpu.async_copy(dinv_hbm.at[dstb[kl]], ddstb[kg], semG[kg])
        pltpu.async_copy(batch_hbm.at[dstb[kl]], bdb[kg], semG[kg])

    def wait_g(kg):
        pltpu.make_async_copy(xw_hbm.at[srcb[0]], rowsb[kg], semG[kg]).wait()
        pltpu.make_async_copy(dinv_hbm.at[srcb[0]], dsrcb[kg],
                              semG[kg]).wait()
        pltpu.make_async_copy(dinv_hbm.at[srcb[0]], ddstb[kg],
                              semG[kg]).wait()
        pltpu.make_async_copy(batch_hbm.at[srcb[0]], bdb[kg], semG[kg]).wait()

    def wait_s(kg):
        pltpu.make_async_copy(rowsb[kg], acca.at[dstb[0]], semS[kg]).wait()

    def wait_w(kw):
        off0 = pl.ds(0, CHUNK)
        pltpu.make_async_copy(normb[kw], outn_hbm.at[off0], semW[kw]).wait()
        pltpu.make_async_copy(fidxb[kw], outf_hbm.at[off0], semW[kw]).wait()

    def process(i, kg, kl, kw):
        rows = rowsb[kg]
        norm = normb[kw]
        fidx = fidxb[kw]

        @pl.when(i >= 2)
        def _():
            wait_w(kw)

        for j in range(CHUNK // 16):
            sl = pl.ds(j * 16, 16)
            norm[sl] = dsrcb[kg][sl] * ewb[kl][sl] * ddstb[kg][sl]
            fidx[sl] = srcb[kl][sl] * G + bdb[kg][sl]

        def scale16(g, cc):
            n16 = norm[pl.ds(g * 16, 16)]
            for l in range(16):
                nv = n16[l]
                e = g * 16 + l
                for j in range(8):
                    fsl = pl.ds(j * 16, 16)
                    rows[e, fsl] = rows[e, fsl] * nv
            return cc

        lax.fori_loop(0, CHUNK // 16, scale16, 0)
        pltpu.async_copy(rows, acca.at[dstb[kl]], semS[kg], add=True)
        off = base_w + i * CHUNK
        pltpu.async_copy(norm, outn_hbm.at[pl.ds(off, CHUNK)], semW[kw])
        pltpu.async_copy(fidx, outf_hbm.at[pl.ds(off, CHUNK)], semW[kw])

    plsc.subcore_barrier()

    issue_l(0, 0)
    issue_l(1, 1)
    issue_l(2, 2)
    wait_l(0)
    issue_g(0, 0, 0)

    UNROLL = 12

    def outer(p, carry):
        for k in range(UNROLL):
            i_static = k
            del i_static
            i = UNROLL * p + k
            kg = k % 3
            kl = k % 4
            kw = k % 2
            kg1 = (k + 1) % 3
            kl1 = (k + 1) % 4

            @pl.when(i + 1 < WCHUNK)
            def _():
                wait_l(kl1)
                issue_g(i + 1, kg1, kl1)

            wait_g(kg)
            process(i, kg, kl, kw)

            @pl.when(i >= 1)
            def _():
                wait_s((k + 2) % 3)

            @pl.when(i + 3 < WCHUNK)
            def _():
                issue_l(i + 3, (k + 3) % 4)

        return carry

    lax.fori_loop(0, WCHUNK // UNROLL, outer, 0)
    wait_s((WCHUNK - 1) % 3)
    wait_w(0)
    wait_w(1)
    plsc.subcore_barrier()

    @pl.when(s == 0)
    def _():
        pltpu.sync_copy(acca, outa_hbm.at[c])


@functools.partial(
    pl.kernel,
    out_type=jax.ShapeDtypeStruct((NC, N * G), jnp.float32),
    mesh=_MESH,
    scratch_types=[
        pltpu.VMEM((WCHUNK, CHUNK), jnp.float32),
        pltpu.VMEM((WCHUNK, CHUNK), jnp.int32),
        pltpu.VMEM_SHARED((N * G,), jnp.float32),
        pltpu.SemaphoreType.DMA,
    ],
)
def _mbuild_kernel(norm_hbm, fidx_hbm, zero_hbm, out_hbm, norm_m, fidx_m,
                   acc, sem):
    c = lax.axis_index("c")
    s = lax.axis_index("s")
    wid = s * NC + c

    pltpu.sync_copy(norm_hbm.at[wid], norm_m)
    pltpu.sync_copy(fidx_hbm.at[wid], fidx_m)

    @pl.when(s == 0)
    def _():
        pltpu.sync_copy(zero_hbm, acc)

    plsc.subcore_barrier()

    def fire(i, carry):
        pltpu.async_copy(norm_m.at[i], acc.at[fidx_m.at[i]], sem, add=True)
        return carry

    lax.fori_loop(0, WCHUNK, fire, 0)

    def drain(i, carry):
        pltpu.make_async_copy(norm_m.at[0], acc.at[fidx_m.at[0]], sem).wait()
        return carry

    lax.fori_loop(0, WCHUNK, drain, 0)
    plsc.subcore_barrier()

    @pl.when(s == 0)
    def _():
        pltpu.sync_copy(acc, out_hbm.at[c])


def _dinv_body(d0_ref, d1_ref, o_ref):
    deg = d0_ref[...] + d1_ref[...]
    safe = jnp.where(deg > 0.0, deg, 1.0)
    o_ref[...] = jnp.where(deg > 0.0, lax.rsqrt(safe), 0.0)


def _mm_body(x_ref, w_ref, o_ref):
    o_ref[...] = jnp.dot(x_ref[...], w_ref[...],
                         preferred_element_type=jnp.float32)


ROWB = 400
NBLK = N // ROWB


def _final_body(p0_ref, p1_ref, b1_ref, m0_ref, m1_ref, bt_ref, w2_ref,
                b2_ref, wl1_ref, bl1_ref, wl2_ref, bl2_ref, o_ref,
                gacc, cnt):
    i = pl.program_id(0)

    @pl.when(i == 0)
    def _():
        gacc[...] = jnp.zeros_like(gacc)
        cnt[...] = jnp.zeros_like(cnt)

    h = jnp.maximum(p0_ref[...] + p1_ref[...] + b1_ref[...], 0.0)
    m = m0_ref[...] + m1_ref[...]
    gacc[...] += lax.dot_general(m, h, (((0,), (0,)), ((), ())),
                                 preferred_element_type=jnp.float32)
    bt = bt_ref[...].reshape(1, ROWB)
    onehot = (lax.broadcasted_iota(jnp.int32, (G, ROWB), 0) == bt)
    cnt[...] += jnp.sum(onehot.astype(jnp.float32), axis=1, keepdims=True)

    @pl.when(i == NBLK - 1)
    def _():
        cc = cnt[...]
        g64 = gacc[...] / jnp.maximum(cc, 1.0)
        mask = jnp.where(cc > 0.0, 1.0, 0.0)
        gp = (jnp.dot(g64, w2_ref[...], preferred_element_type=jnp.float32)
              + mask * b2_ref[...])
        z = (jnp.dot(gp, wl1_ref[...], preferred_element_type=jnp.float32)
             + bl1_ref[...])
        o_ref[...] = (jnp.dot(z, wl2_ref[...],
                              preferred_element_type=jnp.float32)
                      + bl2_ref[...])


def kernel(x, edge_index, edge_attr, batch, W1, b1, W2, b2, Wl1, bl1, Wl2,
           bl2):
    f32 = jnp.float32
    src = edge_index[0]
    dst = edge_index[1]
    loop = jnp.arange(N, dtype=jnp.int32)
    pad = EP - E - N
    src1 = jnp.concatenate([src, loop, jnp.zeros((pad,), jnp.int32)])
    dst1 = jnp.concatenate([dst, loop, jnp.zeros((pad,), jnp.int32)])
    ew1 = jnp.concatenate([edge_attr, jnp.ones((N,), f32),
                           jnp.zeros((pad,), f32)])
    dst3 = dst1.reshape(NW, WCHUNK, CHUNK)
    ew3 = ew1.reshape(NW, WCHUNK, CHUNK)

    z1 = jnp.zeros((N,), f32)
    zrow = jnp.zeros((N, D), f32)
    zm = jnp.zeros((N * G,), f32)

    degp = _deg_kernel(dst3, ew3, z1)

    dinv2d = pl.pallas_call(
        _dinv_body,
        out_shape=jax.ShapeDtypeStruct((80, 125), f32),
    )(degp[0].reshape(80, 125), degp[1].reshape(80, 125))
    dinv = dinv2d.reshape(N)

    xw = pl.pallas_call(
        _mm_body,
        grid=(NBLK,),
        in_specs=[
            pl.BlockSpec((ROWB, D), lambda i: (i, 0)),
            pl.BlockSpec((D, D), lambda i: (0, 0)),
        ],
        out_specs=pl.BlockSpec((ROWB, D), lambda i: (i, 0)),
        out_shape=jax.ShapeDtypeStruct((N, D), f32),
    )(x, W1)

    aggp, normf, fidxf = _agg_kernel(src1, dst1, ew1, dinv, batch, xw, zrow)
    mrawp = _mbuild_kernel(normf.reshape(NW, WCHUNK, CHUNK),
                           fidxf.reshape(NW, WCHUNK, CHUNK), zm)
    m0 = mrawp[0].reshape(N, G)
    m1 = mrawp[1].reshape(N, G)
    bt3 = batch.reshape(NBLK, 1, ROWB)

    out = pl.pallas_call(
        _final_body,
        grid=(NBLK,),
        in_specs=[
            pl.BlockSpec((ROWB, D), lambda i: (i, 0)),
            pl.BlockSpec((ROWB, D), lambda i: (i, 0)),
            pl.BlockSpec((1, D), lambda i: (0, 0)),
            pl.BlockSpec((ROWB, G), lambda i: (i, 0)),
            pl.BlockSpec((ROWB, G), lambda i: (i, 0)),
            pl.BlockSpec((1, 1, ROWB), lambda i: (i, 0, 0)),
            pl.BlockSpec((D, D), lambda i: (0, 0)),
            pl.BlockSpec((1, D), lambda i: (0, 0)),
            pl.BlockSpec((D, 32), lambda i: (0, 0)),
            pl.BlockSpec((1, 32), lambda i: (0, 0)),
            pl.BlockSpec((32, 10), lambda i: (0, 0)),
            pl.BlockSpec((1, 10), lambda i: (0, 0)),
        ],
        out_specs=pl.BlockSpec((G, 10), lambda i: (0, 0)),
        out_shape=jax.ShapeDtypeStruct((G, 10), f32),
        scratch_shapes=[
            pltpu.VMEM((G, D), f32),
            pltpu.VMEM((G, 1), f32),
        ],
    )(aggp[0], aggp[1], b1.reshape(1, D), m0, m1, bt3, W2,
      b2.reshape(1, D), Wl1, bl1.reshape(1, 32), Wl2, bl2.reshape(1, 10))
    return out


# speedup vs baseline: 24.3762x; 2.1469x over previous
import functools

import jax
import jax.numpy as jnp
from jax import lax
from jax.experimental import pallas as pl
from jax.experimental.pallas import tpu as pltpu
from jax.experimental.pallas import tpu_sc as plsc

N = 10000
E = 320000
D = 128
G = 64
NC = 2
NS = 16
NW = NC * NS
CHUNK = 96
WCHUNK = 108
SUBS = 6
SUBSZ = CHUNK // SUBS
PER_W = WCHUNK * CHUNK
EP = NW * PER_W

_MESH = plsc.VectorSubcoreMesh(core_axis_name="c", subcore_axis_name="s")


@functools.partial(
    pl.kernel,
    out_type=jax.ShapeDtypeStruct((NC, N), jnp.float32),
    mesh=_MESH,
    scratch_types=[
        pltpu.VMEM((WCHUNK, CHUNK), jnp.int32),
        pltpu.VMEM((WCHUNK, CHUNK), jnp.float32),
        pltpu.VMEM_SHARED((N,), jnp.float32),
        pltpu.SemaphoreType.DMA,
    ],
)
def _deg_kernel(dst_hbm, ew_hbm, zero_hbm, out_hbm, dst_m, ew_m, acc, sem):
    c = lax.axis_index("c")
    s = lax.axis_index("s")
    wid = s * NC + c

    pltpu.sync_copy(dst_hbm.at[wid], dst_m)
    pltpu.sync_copy(ew_hbm.at[wid], ew_m)

    @pl.when(s == 0)
    def _():
        pltpu.sync_copy(zero_hbm, acc)

    plsc.subcore_barrier()

    def fire(i, carry):
        pltpu.async_copy(ew_m.at[i], acc.at[dst_m.at[i]], sem, add=True)
        return carry

    lax.fori_loop(0, WCHUNK, fire, 0)

    def drain(i, carry):
        pltpu.make_async_copy(ew_m.at[0], acc.at[dst_m.at[0]], sem).wait()
        return carry

    lax.fori_loop(0, WCHUNK, drain, 0)
    plsc.subcore_barrier()

    @pl.when(s == 0)
    def _():
        pltpu.sync_copy(acc, out_hbm.at[c])


@functools.partial(
    pl.kernel,
    out_type=(
        jax.ShapeDtypeStruct((NC, N, D), jnp.float32),
        jax.ShapeDtypeStruct((EP,), jnp.float32),
        jax.ShapeDtypeStruct((EP,), jnp.int32),
    ),
    mesh=_MESH,
    scratch_types=[
        [pltpu.VMEM((CHUNK,), jnp.int32) for _ in range(4)],
        [pltpu.VMEM((CHUNK,), jnp.int32) for _ in range(4)],
        [pltpu.VMEM((CHUNK,), jnp.float32) for _ in range(4)],
        [pltpu.VMEM((CHUNK, D), jnp.float32) for _ in range(3)],
        [pltpu.VMEM((CHUNK,), jnp.float32) for _ in range(3)],
        [pltpu.VMEM((CHUNK,), jnp.float32) for _ in range(3)],
        [pltpu.VMEM((CHUNK,), jnp.int32) for _ in range(3)],
        [[pltpu.VMEM((SUBSZ,), jnp.int32) for _ in range(SUBS)]
         for _ in range(3)],
        [pltpu.VMEM((CHUNK,), jnp.float32) for _ in range(2)],
        [pltpu.VMEM((CHUNK,), jnp.int32) for _ in range(2)],
        pltpu.VMEM_SHARED((N, D), jnp.float32),
        [pltpu.SemaphoreType.DMA for _ in range(4)],
        [pltpu.SemaphoreType.DMA for _ in range(3)],
        [pltpu.SemaphoreType.DMA for _ in range(3)],
        [pltpu.SemaphoreType.DMA for _ in range(3)],
        [pltpu.SemaphoreType.DMA for _ in range(2)],
    ],
)
def _agg_kernel(src_hbm, dst_hbm, ew_hbm, dinv_hbm, batch_hbm, xw_hbm,
                zrow_hbm, outa_hbm, outn_hbm, outf_hbm,
                srcb, dstb, ewb, rowsb, dsrcb, ddstb, bdb, dstq, normb,
                fidxb, acca, semL, semG, semG2, semS, semW):
    c = lax.axis_index("c")
    s = lax.axis_index("s")
    wid = s * NC + c
    base_w = wid * PER_W

    @pl.when(s == 0)
    def _():
        pltpu.sync_copy(zrow_hbm, acca)

    def issue_l(i, kl):
        off = base_w + i * CHUNK
        pltpu.async_copy(src_hbm.at[pl.ds(off, CHUNK)], srcb[kl], semL[kl])
        pltpu.async_copy(dst_hbm.at[pl.ds(off, CHUNK)], dstb[kl], semL[kl])
        pltpu.async_copy(ew_hbm.at[pl.ds(off, CHUNK)], ewb[kl], semL[kl])

    def wait_l(kl):
        off0 = pl.ds(0, CHUNK)
        pltpu.make_async_copy(src_hbm.at[off0], srcb[kl], semL[kl]).wait()
        pltpu.make_async_copy(dst_hbm.at[off0], dstb[kl], semL[kl]).wait()
        pltpu.make_async_copy(ew_hbm.at[off0], ewb[kl], semL[kl]).wait()

    def issue_g(i, kg, kl):
        for q in range(SUBS):
            qsl = pl.ds(q * SUBSZ, SUBSZ)
            pltpu.async_copy(xw_hbm.at[srcb[kl].at[qsl]],
                             rowsb[kg].at[qsl], semG[kg])
        pltpu.async_copy(dinv_hbm.at[srcb[kl]], dsrcb[kg], semG2[kg])
        pltpu.async_copy(dinv_hbm.at[dstb[kl]], ddstb[kg], semG2[kg])
        pltpu.async_copy(batch_hbm.at[dstb[kl]], bdb[kg], semG2[kg])

    def wait_g(kg):
        q0 = pl.ds(0, SUBSZ)
        for q in range(SUBS):
            pltpu.make_async_copy(xw_hbm.at[srcb[0].at[q0]],
                                  rowsb[kg].at[q0], semG[kg]).wait()
        pltpu.make_async_copy(dinv_hbm.at[srcb[0]], dsrcb[kg],
                              semG2[kg]).wait()
        pltpu.make_async_copy(dinv_hbm.at[srcb[0]], ddstb[kg],
                              semG2[kg]).wait()
        pltpu.make_async_copy(batch_hbm.at[srcb[0]], bdb[kg],
                              semG2[kg]).wait()

    def wait_s(kg):
        q0 = pl.ds(0, SUBSZ)
        for q in range(SUBS):
            pltpu.make_async_copy(rowsb[kg].at[q0], acca.at[dstq[kg][q]],
                                  semS[kg]).wait()

    def wait_w(kw):
        off0 = pl.ds(0, CHUNK)
        pltpu.make_async_copy(normb[kw], outn_hbm.at[off0], semW[kw]).wait()
        pltpu.make_async_copy(fidxb[kw], outf_hbm.at[off0], semW[kw]).wait()

    def process(i, kg, kl, kw):
        rows = rowsb[kg]
        norm = normb[kw]
        fidx = fidxb[kw]

        @pl.when(i >= 2)
        def _():
            wait_w(kw)

        for j in range(CHUNK // 16):
            sl = pl.ds(j * 16, 16)
            norm[sl] = dsrcb[kg][sl] * ewb[kl][sl] * ddstb[kg][sl]
            fidx[sl] = srcb[kl][sl] * G + bdb[kg][sl]

        def scale16(g, cc):
            n16 = norm[pl.ds(g * 16, 16)]
            for l in range(16):
                nv = n16[l]
                e = g * 16 + l
                for j in range(8):
                    fsl = pl.ds(j * 16, 16)
                    rows[e, fsl] = rows[e, fsl] * nv
            return cc

        lax.fori_loop(0, CHUNK // 16, scale16, 0)
        for q in range(SUBS):
            dstq[kg][q][...] = dstb[kl][pl.ds(q * SUBSZ, SUBSZ)]
        for q in range(SUBS):
            qsl = pl.ds(q * SUBSZ, SUBSZ)
            pltpu.async_copy(rows.at[qsl], acca.at[dstq[kg][q]], semS[kg],
                             add=True)
        off = base_w + i * CHUNK
        pltpu.async_copy(norm, outn_hbm.at[pl.ds(off, CHUNK)], semW[kw])
        pltpu.async_copy(fidx, outf_hbm.at[pl.ds(off, CHUNK)], semW[kw])

    plsc.subcore_barrier()

    issue_l(0, 0)
    issue_l(1, 1)
    issue_l(2, 2)
    wait_l(0)
    issue_g(0, 0, 0)

    UNROLL = 12

    def outer(p, carry):
        for k in range(UNROLL):
            i = UNROLL * p + k
            kg = k % 3
            kl = k % 4
            kw = k % 2
            kg1 = (k + 1) % 3
            kl1 = (k + 1) % 4

            @pl.when(i >= 2)
            def _():
                wait_s(kg1)

            @pl.when(i + 1 < WCHUNK)
            def _():
                wait_l(kl1)
                issue_g(i + 1, kg1, kl1)

            wait_g(kg)
            process(i, kg, kl, kw)

            @pl.when(i + 3 < WCHUNK)
            def _():
                issue_l(i + 3, (k + 3) % 4)

        return carry

    lax.fori_loop(0, WCHUNK // UNROLL, outer, 0)
    wait_s((WCHUNK - 2) % 3)
    wait_s((WCHUNK - 1) % 3)
    wait_w(0)
    wait_w(1)
    plsc.subcore_barrier()

    @pl.when(s == 0)
    def _():
        pltpu.sync_copy(acca, outa_hbm.at[c])


@functools.partial(
    pl.kernel,
    out_type=jax.ShapeDtypeStruct((NC, N * G), jnp.float32),
    mesh=_MESH,
    scratch_types=[
        pltpu.VMEM((WCHUNK, CHUNK), jnp.float32),
        pltpu.VMEM((WCHUNK, CHUNK), jnp.int32),
        pltpu.VMEM_SHARED((N * G,), jnp.float32),
        pltpu.SemaphoreType.DMA,
    ],
)
def _mbuild_kernel(norm_hbm, fidx_hbm, zero_hbm, out_hbm, norm_m, fidx_m,
                   acc, sem):
    c = lax.axis_index("c")
    s = lax.axis_index("s")
    wid = s * NC + c

    pltpu.sync_copy(norm_hbm.at[wid], norm_m)
    pltpu.sync_copy(fidx_hbm.at[wid], fidx_m)

    @pl.when(s == 0)
    def _():
        pltpu.sync_copy(zero_hbm, acc)

    plsc.subcore_barrier()

    def fire(i, carry):
        pltpu.async_copy(norm_m.at[i], acc.at[fidx_m.at[i]], sem, add=True)
        return carry

    lax.fori_loop(0, WCHUNK, fire, 0)

    def drain(i, carry):
        pltpu.make_async_copy(norm_m.at[0], acc.at[fidx_m.at[0]], sem).wait()
        return carry

    lax.fori_loop(0, WCHUNK, drain, 0)
    plsc.subcore_barrier()

    @pl.when(s == 0)
    def _():
        pltpu.sync_copy(acc, out_hbm.at[c])


def _dinv_body(d0_ref, d1_ref, o_ref):
    deg = d0_ref[...] + d1_ref[...]
    safe = jnp.where(deg > 0.0, deg, 1.0)
    o_ref[...] = jnp.where(deg > 0.0, lax.rsqrt(safe), 0.0)


def _mm_body(x_ref, w_ref, o_ref):
    o_ref[...] = jnp.dot(x_ref[...], w_ref[...],
                         preferred_element_type=jnp.float32)


_RHO = sum([list(range(32 * j, 32 * j + 32, 2))
            + list(range(32 * j + 1, 32 * j + 32, 2))
            for j in range(D // 32)], [])


ROWB = 400
NBLK = N // ROWB


def _final_body(p0_ref, p1_ref, b1_ref, m0_ref, m1_ref, bt_ref, w2_ref,
                b2_ref, wl1_ref, bl1_ref, wl2_ref, bl2_ref, o_ref,
                gacc, cnt):
    i = pl.program_id(0)

    @pl.when(i == 0)
    def _():
        gacc[...] = jnp.zeros_like(gacc)
        cnt[...] = jnp.zeros_like(cnt)

    h = jnp.maximum(p0_ref[...] + p1_ref[...] + b1_ref[...], 0.0)
    m = m0_ref[...] + m1_ref[...]
    gacc[...] += lax.dot_general(m, h, (((0,), (0,)), ((), ())),
                                 preferred_element_type=jnp.float32)
    bt = bt_ref[...].reshape(1, ROWB)
    onehot = (lax.broadcasted_iota(jnp.int32, (G, ROWB), 0) == bt)
    cnt[...] += jnp.sum(onehot.astype(jnp.float32), axis=1, keepdims=True)

    @pl.when(i == NBLK - 1)
    def _():
        cc = cnt[...]
        g64 = gacc[...] / jnp.maximum(cc, 1.0)
        mask = jnp.where(cc > 0.0, 1.0, 0.0)
        gp = (jnp.dot(g64, w2_ref[...], preferred_element_type=jnp.float32)
              + mask * b2_ref[...])
        z = (jnp.dot(gp, wl1_ref[...], preferred_element_type=jnp.float32)
             + bl1_ref[...])
        o_ref[...] = (jnp.dot(z, wl2_ref[...],
                              preferred_element_type=jnp.float32)
                      + bl2_ref[...])


def kernel(x, edge_index, edge_attr, batch, W1, b1, W2, b2, Wl1, bl1, Wl2,
           bl2):
    f32 = jnp.float32
    src = edge_index[0]
    dst = edge_index[1]
    loop = jnp.arange(N, dtype=jnp.int32)
    pad = EP - E - N
    src1 = jnp.concatenate([src, loop, jnp.zeros((pad,), jnp.int32)])
    dst1 = jnp.concatenate([dst, loop, jnp.zeros((pad,), jnp.int32)])
    ew1 = jnp.concatenate([edge_attr, jnp.ones((N,), f32),
                           jnp.zeros((pad,), f32)])
    dst3 = dst1.reshape(NW, WCHUNK, CHUNK)
    ew3 = ew1.reshape(NW, WCHUNK, CHUNK)

    z1 = jnp.zeros((N,), f32)
    zrow = jnp.zeros((N, D), f32)
    zm = jnp.zeros((N * G,), f32)

    degp = _deg_kernel(dst3, ew3, z1)

    dinv2d = pl.pallas_call(
        _dinv_body,
        out_shape=jax.ShapeDtypeStruct((80, 125), f32),
    )(degp[0].reshape(80, 125), degp[1].reshape(80, 125))
    dinv = dinv2d.reshape(N)

    xw = pl.pallas_call(
        _mm_body,
        grid=(NBLK,),
        in_specs=[
            pl.BlockSpec((ROWB, D), lambda i: (i, 0)),
            pl.BlockSpec((D, D), lambda i: (0, 0)),
        ],
        out_specs=pl.BlockSpec((ROWB, D), lambda i: (i, 0)),
        out_shape=jax.ShapeDtypeStruct((N, D), f32),
    )(x, W1)

    aggp, normf, fidxf = _agg_kernel(src1, dst1, ew1, dinv, batch, xw, zrow)
    mrawp = _mbuild_kernel(normf.reshape(NW, WCHUNK, CHUNK),
                           fidxf.reshape(NW, WCHUNK, CHUNK), zm)
    m0 = mrawp[0].reshape(N, G)
    m1 = mrawp[1].reshape(N, G)
    bt3 = batch.reshape(NBLK, 1, ROWB)

    out = pl.pallas_call(
        _final_body,
        grid=(NBLK,),
        in_specs=[
            pl.BlockSpec((ROWB, D), lambda i: (i, 0)),
            pl.BlockSpec((ROWB, D), lambda i: (i, 0)),
            pl.BlockSpec((1, D), lambda i: (0, 0)),
            pl.BlockSpec((ROWB, G), lambda i: (i, 0)),
            pl.BlockSpec((ROWB, G), lambda i: (i, 0)),
            pl.BlockSpec((1, 1, ROWB), lambda i: (i, 0, 0)),
            pl.BlockSpec((D, D), lambda i: (0, 0)),
            pl.BlockSpec((1, D), lambda i: (0, 0)),
            pl.BlockSpec((D, 32), lambda i: (0, 0)),
            pl.BlockSpec((1, 32), lambda i: (0, 0)),
            pl.BlockSpec((32, 10), lambda i: (0, 0)),
            pl.BlockSpec((1, 10), lambda i: (0, 0)),
        ],
        out_specs=pl.BlockSpec((G, 10), lambda i: (0, 0)),
        out_shape=jax.ShapeDtypeStruct((G, 10), f32),
        scratch_shapes=[
            pltpu.VMEM((G, D), f32),
            pltpu.VMEM((G, 1), f32),
        ],
    )(aggp[0], aggp[1], b1[jnp.array(_RHO)].reshape(1, D), m0, m1, bt3,
      W2[jnp.array(_RHO), :], b2.reshape(1, D), Wl1, bl1.reshape(1, 32),
      Wl2, bl2.reshape(1, 10))
    return out
